# baseline (speedup 1.0000x reference)
import functools

import jax
import jax.numpy as jnp
from jax import lax
from jax.experimental import pallas as pl
from jax.experimental.pallas import tpu as pltpu


def _table_kernel(w_ref, b_ref, sos_ref, lab_ref, labt_ref, arg_ref, pred_ref,
                  *, C, T_l, J):
    F = T_l * J
    W = w_ref[...]
    b = b_ref[...]
    sos = sos_ref[...]
    lab = lab_ref[...]
    labt = labt_ref[...]
    lane_t = lax.broadcasted_iota(jnp.int32, (1, F), 1) // J
    cidx1 = lax.broadcasted_iota(jnp.int32, (1, C), 1)
    cidxC = lax.broadcasted_iota(jnp.int32, (C, C), 1)
    ccol = lax.broadcasted_iota(jnp.int32, (C, 1), 0)

    pred_table = jnp.concatenate(
        [jnp.dot(lab[:, t * J:(t + 1) * J], W,
                 preferred_element_type=jnp.float32) + b
         for t in range(T_l)], axis=1)

    lnorms = []
    acc = jnp.zeros((1, C), jnp.float32)
    for t in range(T_l):
        sl = labt[t * J:(t + 1) * J, :]
        acc = acc + jnp.sum(sl * sl, axis=0, keepdims=True)
        lnorms.append(acc)

    def classify(p, t, cidx):
        pm = jnp.where(lane_t < t, p, 0.0)
        pd = jnp.dot(pm, labt, preferred_element_type=jnp.float32)
        score = lnorms[t - 1] - 2.0 * pd
        minv = jnp.min(score, axis=-1, keepdims=True)
        return jnp.min(jnp.where(score == minv, cidx, C),
                       axis=-1, keepdims=True)

    onehots = []
    for t in range(1, T_l + 1):
        arg = classify(pred_table, t, cidxC)
        onehots.append((cidxC == arg).astype(jnp.float32))

    p0row = jnp.dot(sos, W, preferred_element_type=jnp.float32) + b
    p0 = jnp.concatenate(
        [p0row] + [jnp.zeros((1, J), jnp.float32)] * (T_l - 1), axis=1)
    arg0 = classify(p0, 1, cidx1)
    oh = (cidx1 == arg0).astype(jnp.float32)

    for s in range(1, T_l):
        oh = jnp.dot(oh, onehots[s - 1], preferred_element_type=jnp.float32)

    oh_fin = jnp.dot(oh, onehots[T_l - 1], preferred_element_type=jnp.float32)
    arg_fin = jnp.min(jnp.where(oh_fin > 0.5, cidx1, C),
                      axis=-1, keepdims=True)

    arg7 = jnp.min(jnp.where(oh > 0.5, cidx1, C),
                   axis=-1, keepdims=True)
    rowmask = (ccol == arg7).astype(jnp.float32)
    pred_fin = jnp.sum(pred_table * rowmask, axis=0, keepdims=True)

    arg_ref[...] = jnp.broadcast_to(arg_fin, arg_ref.shape)
    pred_ref[...] = jnp.broadcast_to(pred_fin, pred_ref.shape)


def kernel(x, lens, W, b, sos, label_seqs):
    B = x.shape[0]
    C, T_l, J = label_seqs.shape
    F = T_l * J

    lab = label_seqs.astype(jnp.float32).reshape(C, F)
    labt = lab.T
    b2 = b.astype(jnp.float32).reshape(1, J)
    sos2 = sos.astype(jnp.float32).reshape(1, J)
    Wf = W.astype(jnp.float32)

    if B >= 16 and B % 16 == 0:
        bb = B // 2
    else:
        bb = B
    n_blk = B // bb

    kern = functools.partial(_table_kernel, C=C, T_l=T_l, J=J)
    arg_out, pred_out = pl.pallas_call(
        kern,
        out_shape=(jax.ShapeDtypeStruct((B, 128), jnp.int32),
                   jax.ShapeDtypeStruct((B, F), jnp.float32)),
        grid=(n_blk,),
        in_specs=[
            pl.BlockSpec((J, J), lambda i: (0, 0)),
            pl.BlockSpec((1, J), lambda i: (0, 0)),
            pl.BlockSpec((1, J), lambda i: (0, 0)),
            pl.BlockSpec((C, F), lambda i: (0, 0)),
            pl.BlockSpec((F, C), lambda i: (0, 0)),
        ],
        out_specs=(pl.BlockSpec((bb, 128), lambda i: (i, 0)),
                   pl.BlockSpec((bb, F), lambda i: (i, 0))),
        compiler_params=pltpu.CompilerParams(
            dimension_semantics=("parallel",)),
    )(Wf, b2, sos2, lab, labt)

    pred_label_sofar = arg_out[:, 0]
    pred_label_seq = pred_out.reshape(B, T_l, J)
    return pred_label_sofar, pred_label_seq


# speedup vs baseline: 1.3532x; 1.3532x over previous
import functools

import jax
import jax.numpy as jnp
from jax import lax
from jax.experimental import pallas as pl
from jax.experimental.pallas import tpu as pltpu


def _table_kernel(w_ref, b_ref, sos_ref, lab_ref, arg_ref, pred_ref,
                  *, C, T_l, J):
    F = T_l * J
    W = w_ref[...]
    b = b_ref[...]
    sos = sos_ref[...]
    lab = lab_ref[...]
    labt = jnp.transpose(lab)
    lane_t = lax.broadcasted_iota(jnp.int32, (1, F), 1) // J
    cidx1 = lax.broadcasted_iota(jnp.int32, (1, C), 1)
    cidxC = lax.broadcasted_iota(jnp.int32, (C, C), 1)
    ccol = lax.broadcasted_iota(jnp.int32, (C, 1), 0)

    pred_table = jnp.concatenate(
        [jnp.dot(lab[:, t * J:(t + 1) * J], W,
                 preferred_element_type=jnp.float32) + b
         for t in range(T_l)], axis=1)

    lnorms = []
    acc = jnp.zeros((1, C), jnp.float32)
    for t in range(T_l):
        sl = labt[t * J:(t + 1) * J, :]
        acc = acc + jnp.sum(sl * sl, axis=0, keepdims=True)
        lnorms.append(acc)

    def classify(p, t, cidx):
        pm = jnp.where(lane_t < t, p, 0.0)
        pd = jnp.dot(pm, labt, preferred_element_type=jnp.float32)
        score = lnorms[t - 1] - 2.0 * pd
        minv = jnp.min(score, axis=-1, keepdims=True)
        return jnp.min(jnp.where(score == minv, cidx, C),
                       axis=-1, keepdims=True)

    onehots = []
    for t in range(1, T_l + 1):
        arg = classify(pred_table, t, cidxC)
        onehots.append((cidxC == arg).astype(jnp.float32))

    p0row = jnp.dot(sos, W, preferred_element_type=jnp.float32) + b
    p0 = jnp.concatenate(
        [p0row] + [jnp.zeros((1, J), jnp.float32)] * (T_l - 1), axis=1)
    arg0 = classify(p0, 1, cidx1)
    oh = (cidx1 == arg0).astype(jnp.float32)

    for s in range(1, T_l):
        oh = jnp.dot(oh, onehots[s - 1], preferred_element_type=jnp.float32)

    oh_fin = jnp.dot(oh, onehots[T_l - 1], preferred_element_type=jnp.float32)
    arg_fin = jnp.min(jnp.where(oh_fin > 0.5, cidx1, C),
                      axis=-1, keepdims=True)

    arg7 = jnp.min(jnp.where(oh > 0.5, cidx1, C),
                   axis=-1, keepdims=True)
    rowmask = (ccol == arg7).astype(jnp.float32)
    pred_fin = jnp.sum(pred_table * rowmask, axis=0, keepdims=True)

    arg_ref[...] = jnp.broadcast_to(arg_fin, arg_ref.shape)
    pred_ref[...] = pred_fin


def kernel(x, lens, W, b, sos, label_seqs):
    B = x.shape[0]
    C, T_l, J = label_seqs.shape
    F = T_l * J

    lab = label_seqs.astype(jnp.float32).reshape(C, F)
    b2 = b.astype(jnp.float32).reshape(1, J)
    sos2 = sos.astype(jnp.float32).reshape(1, J)
    Wf = W.astype(jnp.float32)

    kern = functools.partial(_table_kernel, C=C, T_l=T_l, J=J)
    arg_out, pred_out = pl.pallas_call(
        kern,
        out_shape=(jax.ShapeDtypeStruct((1, C), jnp.int32),
                   jax.ShapeDtypeStruct((1, F), jnp.float32)),
        grid=(1,),
        in_specs=[
            pl.BlockSpec((J, J), lambda i: (0, 0)),
            pl.BlockSpec((1, J), lambda i: (0, 0)),
            pl.BlockSpec((1, J), lambda i: (0, 0)),
            pl.BlockSpec((C, F), lambda i: (0, 0)),
        ],
        out_specs=(pl.BlockSpec((1, C), lambda i: (0, 0)),
                   pl.BlockSpec((1, F), lambda i: (0, 0))),
        compiler_params=pltpu.CompilerParams(
            dimension_semantics=("arbitrary",)),
    )(Wf, b2, sos2, lab)

    pred_label_sofar = jnp.broadcast_to(arg_out[0, 0], (B,))
    pred_label_seq = jnp.broadcast_to(pred_out.reshape(1, T_l, J), (B, T_l, J))
    return pred_label_sofar, pred_label_seq


# speedup vs baseline: 1.3971x; 1.0325x over previous
import functools

import jax
import jax.numpy as jnp
from jax import lax
from jax.experimental import pallas as pl
from jax.experimental.pallas import tpu as pltpu


def _table_kernel(w_ref, b_ref, sos_ref, lab_ref, arg_ref, pred_ref,
                  *, C, T_l, J):
    F = T_l * J
    W = w_ref[...]
    b = b_ref[...]
    sos = sos_ref[...]
    lab = lab_ref[...]
    labt = jnp.transpose(lab)
    lane_t = lax.broadcasted_iota(jnp.int32, (1, F), 1) // J
    cidx1 = lax.broadcasted_iota(jnp.int32, (1, C), 1)
    cidxC = lax.broadcasted_iota(jnp.int32, (C, C), 1)
    ccol = lax.broadcasted_iota(jnp.int32, (C, 1), 0)

    pred_table = jnp.concatenate(
        [jnp.dot(lab[:, t * J:(t + 1) * J], W,
                 preferred_element_type=jnp.float32) + b
         for t in range(T_l)], axis=1)

    lnorms = []
    acc = jnp.zeros((1, C), jnp.float32)
    for t in range(T_l):
        sl = labt[t * J:(t + 1) * J, :]
        acc = acc + jnp.sum(sl * sl, axis=0, keepdims=True)
        lnorms.append(acc)

    def classify(p, t, cidx):
        pm = jnp.where(lane_t < t, p, 0.0)
        pd = jnp.dot(pm, labt, preferred_element_type=jnp.float32)
        score = lnorms[t - 1] - 2.0 * pd
        minv = jnp.min(score, axis=-1, keepdims=True)
        return jnp.min(jnp.where(score == minv, cidx, C),
                       axis=-1, keepdims=True)

    onehots = []
    for t in range(1, T_l + 1):
        arg = classify(pred_table, t, cidxC)
        onehots.append((cidxC == arg).astype(jnp.float32))

    p0row = jnp.dot(sos, W, preferred_element_type=jnp.float32) + b
    p0 = jnp.concatenate(
        [p0row] + [jnp.zeros((1, J), jnp.float32)] * (T_l - 1), axis=1)
    arg0 = classify(p0, 1, cidx1)
    oh = (cidx1 == arg0).astype(jnp.float32)

    for s in range(1, T_l):
        oh = jnp.dot(oh, onehots[s - 1], preferred_element_type=jnp.float32)

    oh_fin = jnp.dot(oh, onehots[T_l - 1], preferred_element_type=jnp.float32)
    arg_fin = jnp.min(jnp.where(oh_fin > 0.5, cidx1, C),
                      axis=-1, keepdims=True)

    arg7 = jnp.min(jnp.where(oh > 0.5, cidx1, C),
                   axis=-1, keepdims=True)
    rowmask = (ccol == arg7).astype(jnp.float32)
    pred_fin = jnp.sum(pred_table * rowmask, axis=0, keepdims=True)

    arg_ref[...] = jnp.broadcast_to(arg_fin, arg_ref.shape)
    pred_ref[...] = jnp.broadcast_to(pred_fin, pred_ref.shape)


def kernel(x, lens, W, b, sos, label_seqs):
    B = x.shape[0]
    C, T_l, J = label_seqs.shape
    F = T_l * J

    lab = label_seqs.astype(jnp.float32).reshape(C, F)
    b2 = b.astype(jnp.float32).reshape(1, J)
    sos2 = sos.astype(jnp.float32).reshape(1, J)
    Wf = W.astype(jnp.float32)

    kern = functools.partial(_table_kernel, C=C, T_l=T_l, J=J)
    arg_out, pred_out = pl.pallas_call(
        kern,
        out_shape=(jax.ShapeDtypeStruct((1, B), jnp.int32),
                   jax.ShapeDtypeStruct((B, F), jnp.float32)),
        grid=(1,),
        in_specs=[
            pl.BlockSpec((J, J), lambda i: (0, 0)),
            pl.BlockSpec((1, J), lambda i: (0, 0)),
            pl.BlockSpec((1, J), lambda i: (0, 0)),
            pl.BlockSpec((C, F), lambda i: (0, 0)),
        ],
        out_specs=(pl.BlockSpec((1, B), lambda i: (0, 0)),
                   pl.BlockSpec((B, F), lambda i: (0, 0))),
        compiler_params=pltpu.CompilerParams(
            dimension_semantics=("arbitrary",)),
    )(Wf, b2, sos2, lab)

    pred_label_sofar = arg_out.reshape(B)
    pred_label_seq = pred_out.reshape(B, T_l, J)
    return pred_label_sofar, pred_label_seq


# speedup vs baseline: 1.4081x; 1.0079x over previous
import functools

import jax
import jax.numpy as jnp
from jax import lax
from jax.experimental import pallas as pl
from jax.experimental.pallas import tpu as pltpu


def _table_kernel(w_ref, b_ref, sos_ref, lab_ref, arg_ref, pred_ref,
                  *, C, T_l, J):
    F = T_l * J
    W = w_ref[...]
    b = b_ref[...]
    sos = sos_ref[...]
    lab = lab_ref[...]
    labt = jnp.transpose(lab)
    lane_t = lax.broadcasted_iota(jnp.int32, (1, F), 1) // J
    cidx1 = lax.broadcasted_iota(jnp.int32, (1, C), 1)
    cidxC = lax.broadcasted_iota(jnp.int32, (C, C), 1)
    ccol = lax.broadcasted_iota(jnp.int32, (C, 1), 0)

    pred_table = jnp.concatenate(
        [jnp.dot(lab[:, t * J:(t + 1) * J], W,
                 preferred_element_type=jnp.float32) + b
         for t in range(T_l)], axis=1)

    lnorms = []
    acc = jnp.zeros((1, C), jnp.float32)
    for t in range(T_l):
        sl = labt[t * J:(t + 1) * J, :]
        acc = acc + jnp.sum(sl * sl, axis=0, keepdims=True)
        lnorms.append(acc)

    def classify(p, t, cidx):
        pm = jnp.where(lane_t < t, p, 0.0)
        pd = jnp.dot(pm, labt, preferred_element_type=jnp.float32)
        score = lnorms[t - 1] - 2.0 * pd
        minv = jnp.min(score, axis=-1, keepdims=True)
        return jnp.min(jnp.where(score == minv, cidx, C),
                       axis=-1, keepdims=True)

    q = jnp.concatenate(
        [jnp.where(lane_t < t, pred_table, 0.0) for t in range(1, T_l + 1)],
        axis=0)
    pd_all = jnp.dot(q, labt, preferred_element_type=jnp.float32)
    lnorm_all = jnp.concatenate(
        [jnp.broadcast_to(lnorms[t], (C, C)) for t in range(T_l)], axis=0)
    score_all = lnorm_all - 2.0 * pd_all
    minv_all = jnp.min(score_all, axis=-1, keepdims=True)
    cidx_all = lax.broadcasted_iota(jnp.int32, (T_l * C, C), 1)
    arg_all = jnp.min(jnp.where(score_all == minv_all, cidx_all, C),
                      axis=-1, keepdims=True)

    onehots = [
        (cidxC == arg_all[t * C:(t + 1) * C, :]).astype(jnp.float32)
        for t in range(T_l)]

    p0row = jnp.dot(sos, W, preferred_element_type=jnp.float32) + b
    p0 = jnp.concatenate(
        [p0row] + [jnp.zeros((1, J), jnp.float32)] * (T_l - 1), axis=1)
    arg0 = classify(p0, 1, cidx1)
    oh = (cidx1 == arg0).astype(jnp.float32)

    for s in range(1, T_l):
        oh = jnp.dot(oh, onehots[s - 1], preferred_element_type=jnp.float32)

    oh_fin = jnp.dot(oh, onehots[T_l - 1], preferred_element_type=jnp.float32)
    arg_fin = jnp.min(jnp.where(oh_fin > 0.5, cidx1, C),
                      axis=-1, keepdims=True)

    arg7 = jnp.min(jnp.where(oh > 0.5, cidx1, C),
                   axis=-1, keepdims=True)
    rowmask = (ccol == arg7).astype(jnp.float32)
    pred_fin = jnp.sum(pred_table * rowmask, axis=0, keepdims=True)

    arg_ref[...] = jnp.broadcast_to(arg_fin, arg_ref.shape)
    pred_ref[...] = jnp.broadcast_to(pred_fin, pred_ref.shape)


def kernel(x, lens, W, b, sos, label_seqs):
    B = x.shape[0]
    C, T_l, J = label_seqs.shape
    F = T_l * J

    lab = label_seqs.astype(jnp.float32).reshape(C, F)
    b2 = b.astype(jnp.float32).reshape(1, J)
    sos2 = sos.astype(jnp.float32).reshape(1, J)
    Wf = W.astype(jnp.float32)

    kern = functools.partial(_table_kernel, C=C, T_l=T_l, J=J)
    arg_out, pred_out = pl.pallas_call(
        kern,
        out_shape=(jax.ShapeDtypeStruct((1, B), jnp.int32),
                   jax.ShapeDtypeStruct((B, F), jnp.float32)),
        grid=(1,),
        in_specs=[
            pl.BlockSpec((J, J), lambda i: (0, 0)),
            pl.BlockSpec((1, J), lambda i: (0, 0)),
            pl.BlockSpec((1, J), lambda i: (0, 0)),
            pl.BlockSpec((C, F), lambda i: (0, 0)),
        ],
        out_specs=(pl.BlockSpec((1, B), lambda i: (0, 0)),
                   pl.BlockSpec((B, F), lambda i: (0, 0))),
        compiler_params=pltpu.CompilerParams(
            dimension_semantics=("arbitrary",)),
    )(Wf, b2, sos2, lab)

    pred_label_sofar = arg_out.reshape(B)
    pred_label_seq = pred_out.reshape(B, T_l, J)
    return pred_label_sofar, pred_label_seq
